# baseline (speedup 1.0000x reference)
import functools
import math

import jax
import jax.numpy as jnp
from jax import lax
from jax.experimental import pallas as pl
from jax.experimental.pallas import tpu as pltpu
from jax.experimental.pallas import tpu_sc as plsc

_D = 768
_SEQ = 2048
_BATCH = 4
_TOTAL = _BATCH * _SEQ
_NC, _NS = 2, 16
_NW = _NC * _NS
_B_PER_W = _TOTAL // _NW
_S_PER_W = _SEQ // _NW
_Q = 8
_KC = _BATCH * _Q
_NCHUNK = _S_PER_W // _Q
_NB = 3
_LANES = 16
_VPR = _D // _LANES
_SCALE = math.sqrt(float(_D))

_mesh = plsc.VectorSubcoreMesh(
    core_axis_name="c", subcore_axis_name="s", num_cores=_NC, num_subcores=_NS
)


@functools.partial(
    pl.kernel,
    out_type=jax.ShapeDtypeStruct((_TOTAL, _D), jnp.float32),
    mesh=_mesh,
    scratch_types=[
        pltpu.VMEM((_B_PER_W,), jnp.int32),
        pltpu.VMEM((_S_PER_W, _D), jnp.float32),
        [pltpu.VMEM((_KC, _D), jnp.float32) for _ in range(_NB)],
        pltpu.SemaphoreType.DMA,
        [pltpu.SemaphoreType.DMA for _ in range(_NB)],
        [pltpu.SemaphoreType.DMA for _ in range(_NB)],
    ],
)
def _embed(
    x_hbm, pos_hbm, table_hbm, out_hbm,
    idx_v, pos_res, gbufs, psem, gsems, osems,
):
    cid = lax.axis_index("c")
    sid = lax.axis_index("s")
    wid = cid * _NS + sid
    p0 = wid * _S_PER_W

    pos_load = pltpu.async_copy(
        pos_hbm.at[0, pl.ds(p0, _S_PER_W)], pos_res, psem
    )
    pltpu.sync_copy(x_hbm.at[pl.ds(wid * _B_PER_W, _B_PER_W)], idx_v)

    gathers = [None] * _NB
    outs = [[None] * _BATCH for _ in range(_NB)]

    def issue_gather(c):
        g = c % _NB
        gathers[g] = pltpu.async_copy(
            table_hbm.at[idx_v.at[pl.ds(c * _KC, _KC)]], gbufs[g], gsems[g]
        )

    for c in range(_NB):
        issue_gather(c)
    pos_load.wait()

    for c in range(_NCHUNK):
        g = c % _NB
        gathers[g].wait()
        gbuf = gbufs[g]
        off = c * _Q

        @pl.loop(0, _Q, unroll=2)
        def _(q):
            for j in range(_VPR):
                sl = pl.ds(j * _LANES, _LANES)
                vpos = pos_res[off + q, sl]
                for b in range(_BATCH):
                    gbuf[b * _Q + q, sl] = gbuf[b * _Q + q, sl] * _SCALE + vpos

        for b in range(_BATCH):
            outs[g][b] = pltpu.async_copy(
                gbuf.at[pl.ds(b * _Q, _Q)],
                out_hbm.at[pl.ds(b * _SEQ + p0 + off, _Q)],
                osems[g],
            )
        if c >= 1:
            pg = (c - 1) % _NB
            for b in range(_BATCH):
                outs[pg][b].wait()
                outs[pg][b] = None
            if c + 2 < _NCHUNK:
                issue_gather(c + 2)

    for ring in outs:
        for o in ring:
            if o is not None:
                o.wait()


def kernel(x, token_table, pos_embedding):
    x_shuf = (
        x.reshape(_BATCH, _NW, _NCHUNK, _Q)
        .transpose(1, 2, 0, 3)
        .reshape(_TOTAL)
        .astype(jnp.int32)
    )
    out = _embed(x_shuf, pos_embedding, token_table)
    return out.reshape(_BATCH, _SEQ, _D)


# speedup vs baseline: 1.0057x; 1.0057x over previous
import functools
import math

import jax
import jax.numpy as jnp
from jax import lax
from jax.experimental import pallas as pl
from jax.experimental.pallas import tpu as pltpu
from jax.experimental.pallas import tpu_sc as plsc

_D = 768
_SEQ = 2048
_BATCH = 4
_TOTAL = _BATCH * _SEQ
_NC, _NS = 2, 16
_NW = _NC * _NS
_B_PER_W = _TOTAL // _NW
_S_PER_W = _SEQ // _NW
_Q = 8
_KC = _BATCH * _Q
_NCHUNK = _S_PER_W // _Q
_NB = 3
_LANES = 16
_VPR = _D // _LANES
_SCALE = math.sqrt(float(_D))

_mesh = plsc.VectorSubcoreMesh(
    core_axis_name="c", subcore_axis_name="s", num_cores=_NC, num_subcores=_NS
)


@functools.partial(
    pl.kernel,
    out_type=jax.ShapeDtypeStruct((_TOTAL, _D), jnp.float32),
    mesh=_mesh,
    scratch_types=[
        pltpu.VMEM((_B_PER_W,), jnp.int32),
        pltpu.VMEM((_S_PER_W, _D), jnp.float32),
        pltpu.VMEM((_NB * _KC, _D), jnp.float32),
        pltpu.SemaphoreType.DMA,
        [pltpu.SemaphoreType.DMA for _ in range(_NB)],
    ],
)
def _embed(
    x_hbm, pos_hbm, table_hbm, out_hbm,
    idx_v, pos_res, gball, psem, sems,
):
    cid = lax.axis_index("c")
    sid = lax.axis_index("s")
    wid = cid * _NS + sid
    p0 = wid * _S_PER_W

    pos_load = pltpu.async_copy(
        pos_hbm.at[0, pl.ds(p0, _S_PER_W)], pos_res, psem
    )
    pltpu.sync_copy(x_hbm.at[pl.ds(wid * _B_PER_W, _B_PER_W)], idx_v)

    gathers = [None] * _NB
    outs = [[None] * _BATCH for _ in range(_NB)]

    def issue_gather(c):
        g = c % _NB
        gathers[g] = pltpu.async_copy(
            table_hbm.at[idx_v.at[pl.ds(c * _KC, _KC)]],
            gball.at[pl.ds(g * _KC, _KC)],
            sems[g],
        )

    for c in range(_NB):
        issue_gather(c)
    pos_load.wait()

    for c in range(_NCHUNK):
        g = c % _NB
        gathers[g].wait()
        row0 = g * _KC
        off = c * _Q

        @pl.loop(0, _Q, unroll=1)
        def _(q):
            for j in range(_VPR):
                sl = pl.ds(j * _LANES, _LANES)
                vpos = pos_res[off + q, sl]
                for b in range(_BATCH):
                    r = row0 + b * _Q + q
                    gball[r, sl] = gball[r, sl] * _SCALE + vpos

        for b in range(_BATCH):
            outs[g][b] = pltpu.async_copy(
                gball.at[pl.ds(row0 + b * _Q, _Q)],
                out_hbm.at[pl.ds(b * _SEQ + p0 + off, _Q)],
                sems[g],
            )
        if c >= 1:
            pg = (c - 1) % _NB
            for b in range(_BATCH):
                outs[pg][b].wait()
                outs[pg][b] = None
            if c + 2 < _NCHUNK:
                issue_gather(c + 2)

    for ring in outs:
        for o in ring:
            if o is not None:
                o.wait()


def kernel(x, token_table, pos_embedding):
    x_shuf = (
        x.reshape(_BATCH, _NW, _NCHUNK, _Q)
        .transpose(1, 2, 0, 3)
        .reshape(_TOTAL)
        .astype(jnp.int32)
    )
    out = _embed(x_shuf, pos_embedding, token_table)
    return out.reshape(_BATCH, _SEQ, _D)
